# Initial kernel scaffold; baseline (speedup 1.0000x reference)
import jax
import jax.numpy as jnp
from jax.experimental import pallas as pl


def kernel(x, embedding_table, pos_table):
    raise NotImplementedError("write your pallas kernel here")


# speedup vs baseline: 3.4805x; 3.4805x over previous
import functools

import jax
import jax.numpy as jnp
from jax import lax
from jax.experimental import pallas as pl
from jax.experimental.pallas import tpu as pltpu
from jax.experimental.pallas import tpu_sc as plsc

NC, NS, LANES = 2, 16, 16
NW = NC * NS
CH = 2


def kernel(x, embedding_table, pos_table):
    B, L = x.shape
    V, D = embedding_table.shape
    rows = B * L
    rows_per_w = rows // NW
    seqs_per_w = rows_per_w // L
    n_chunks = seqs_per_w // CH
    chunk_rows = CH * L

    idx_flat = x.reshape(rows).astype(jnp.int32)

    mesh = plsc.VectorSubcoreMesh(
        core_axis_name="c", subcore_axis_name="s",
        num_cores=NC, num_subcores=NS,
    )

    @functools.partial(
        pl.kernel,
        out_type=jax.ShapeDtypeStruct((rows, D), jnp.float32),
        mesh=mesh,
        scratch_types=[
            pltpu.VMEM((rows_per_w,), jnp.int32),
            pltpu.VMEM((L, D), jnp.float32),
            pltpu.VMEM((chunk_rows, D), jnp.float32),
            pltpu.SemaphoreType.DMA,
        ],
        compiler_params=pltpu.CompilerParams(use_tc_tiling_on_sc=False),
    )
    def emb_kernel(idx_hbm, table_hbm, pos_hbm, out_hbm, idx_v, pos_v, rows_v, sem):
        wid = lax.axis_index("s") * NC + lax.axis_index("c")
        base = wid * rows_per_w
        pltpu.sync_copy(idx_hbm.at[pl.ds(base, rows_per_w)], idx_v)
        pltpu.sync_copy(pos_hbm, pos_v)

        def chunk_body(i, carry):
            off = i * chunk_rows
            pltpu.async_copy(
                table_hbm.at[idx_v.at[pl.ds(off, chunk_rows)]], rows_v, sem
            ).wait()

            def add_body(l, c2):
                for s_i in range(CH):
                    r = s_i * L + l
                    for c4 in range(D // LANES):
                        col = c4 * LANES
                        pvec = pos_v[l, pl.ds(col, LANES)]
                        plsc.addupdate(rows_v.at[r, pl.ds(col, LANES)], pvec)
                return c2

            lax.fori_loop(0, L, add_body, 0)
            pltpu.sync_copy(rows_v, out_hbm.at[pl.ds(base + off, chunk_rows)])
            return carry

        lax.fori_loop(0, n_chunks, chunk_body, 0)

    out = emb_kernel(idx_flat, embedding_table, pos_table)
    return out.reshape(B, L, D)


# speedup vs baseline: 4.0140x; 1.1533x over previous
import functools

import jax
import jax.numpy as jnp
from jax import lax
from jax.experimental import pallas as pl
from jax.experimental.pallas import tpu as pltpu
from jax.experimental.pallas import tpu_sc as plsc

NC, NS, LANES = 2, 16, 16
NW = NC * NS
CH = 2


def kernel(x, embedding_table, pos_table):
    B, L = x.shape
    V, D = embedding_table.shape
    rows = B * L
    rows_per_w = rows // NW
    seqs_per_w = rows_per_w // L
    n_chunks = seqs_per_w // CH
    chunk_rows = CH * L

    idx_flat = x.reshape(rows).astype(jnp.int32)

    mesh = plsc.VectorSubcoreMesh(
        core_axis_name="c", subcore_axis_name="s",
        num_cores=NC, num_subcores=NS,
    )

    @functools.partial(
        pl.kernel,
        out_type=jax.ShapeDtypeStruct((rows, D), jnp.float32),
        mesh=mesh,
        scratch_types=[
            pltpu.VMEM((rows_per_w,), jnp.int32),
            pltpu.VMEM((L, D), jnp.float32),
            pltpu.VMEM((chunk_rows, D), jnp.float32),
            pltpu.VMEM((chunk_rows, D), jnp.float32),
            pltpu.SemaphoreType.DMA,
            pltpu.SemaphoreType.DMA,
            pltpu.SemaphoreType.DMA,
            pltpu.SemaphoreType.DMA,
        ],
        compiler_params=pltpu.CompilerParams(use_tc_tiling_on_sc=False),
    )
    def emb_kernel(idx_hbm, table_hbm, pos_hbm, out_hbm, idx_v, pos_v,
                   buf0, buf1, gsem0, gsem1, wsem0, wsem1):
        wid = lax.axis_index("s") * NC + lax.axis_index("c")
        base = wid * rows_per_w
        pltpu.sync_copy(idx_hbm.at[pl.ds(base, rows_per_w)], idx_v)
        pltpu.sync_copy(pos_hbm, pos_v)

        bufs = (buf0, buf1)
        gsems = (gsem0, gsem1)
        wsems = (wsem0, wsem1)

        def start_gather(chunk, b):
            off = chunk * chunk_rows
            pltpu.async_copy(
                table_hbm.at[idx_v.at[pl.ds(off, chunk_rows)]], bufs[b], gsems[b])

        def wait_gather(b):
            pltpu.make_async_copy(
                table_hbm.at[idx_v.at[pl.ds(0, chunk_rows)]], bufs[b], gsems[b]
            ).wait()

        def start_wb(chunk, b):
            off = chunk * chunk_rows
            pltpu.async_copy(
                bufs[b], out_hbm.at[pl.ds(base + off, chunk_rows)], wsems[b])

        def wait_wb(b):
            pltpu.make_async_copy(
                bufs[b], out_hbm.at[pl.ds(0, chunk_rows)], wsems[b]).wait()

        def add_pos(b):
            buf = bufs[b]

            def add_body(l, c2):
                for s_i in range(CH):
                    r = s_i * L + l
                    for c4 in range(D // LANES):
                        col = c4 * LANES
                        plsc.addupdate(buf.at[r, pl.ds(col, LANES)],
                                       pos_v[l, pl.ds(col, LANES)])
                return c2

            lax.fori_loop(0, L, add_body, 0, unroll=2)

        start_gather(0, 0)
        start_gather(1, 1)
        wait_gather(0)
        add_pos(0)
        start_wb(0, 0)

        def outer(io, carry):
            for k in range(2):
                chunk = 2 * io + 1 + k
                b = (1 + k) % 2
                wait_gather(b)
                wait_wb(1 - b)
                start_gather(chunk + 1, 1 - b)
                add_pos(b)
                start_wb(chunk, b)
            return carry

        lax.fori_loop(0, (n_chunks - 2) // 2, outer, 0)

        wait_gather(1)
        wait_wb(0)
        add_pos(1)
        start_wb(n_chunks - 1, 1)
        wait_wb(1)

    out = emb_kernel(idx_flat, embedding_table, pos_table)
    return out.reshape(B, L, D)


# speedup vs baseline: 4.0145x; 1.0001x over previous
import functools

import jax
import jax.numpy as jnp
from jax import lax
from jax.experimental import pallas as pl
from jax.experimental.pallas import tpu as pltpu
from jax.experimental.pallas import tpu_sc as plsc

NC, NS, LANES = 2, 16, 16
NW = NC * NS
CH = 2


def kernel(x, embedding_table, pos_table):
    B, L = x.shape
    V, D = embedding_table.shape
    seqs_per_w = B // NW
    n_chunks = seqs_per_w // CH
    chunk_rows = CH * L

    x = x.astype(jnp.int32)

    mesh = plsc.VectorSubcoreMesh(
        core_axis_name="c", subcore_axis_name="s",
        num_cores=NC, num_subcores=NS,
    )

    @functools.partial(
        pl.kernel,
        out_type=jax.ShapeDtypeStruct((B, L, D), jnp.float32),
        mesh=mesh,
        scratch_types=[
            pltpu.VMEM((seqs_per_w, L), jnp.int32),
            pltpu.VMEM((L, D), jnp.float32),
            pltpu.VMEM((CH, L, D), jnp.float32),
            pltpu.VMEM((CH, L, D), jnp.float32),
            pltpu.SemaphoreType.DMA,
            pltpu.SemaphoreType.DMA,
            pltpu.SemaphoreType.DMA,
            pltpu.SemaphoreType.DMA,
        ],
        compiler_params=pltpu.CompilerParams(use_tc_tiling_on_sc=False),
    )
    def emb_kernel(idx_hbm, table_hbm, pos_hbm, out_hbm, idx_v, pos_v,
                   buf0, buf1, gsem0, gsem1, wsem0, wsem1):
        wid = lax.axis_index("s") * NC + lax.axis_index("c")
        seq_base = wid * seqs_per_w
        pltpu.sync_copy(idx_hbm.at[pl.ds(seq_base, seqs_per_w)], idx_v)
        pltpu.sync_copy(pos_hbm, pos_v)

        bufs = (buf0, buf1)
        gsems = (gsem0, gsem1)
        wsems = (wsem0, wsem1)

        def start_gather(chunk, b):
            for s_i in range(CH):
                pltpu.async_copy(
                    table_hbm.at[idx_v.at[chunk * CH + s_i]],
                    bufs[b].at[s_i], gsems[b])

        def wait_gather(b):
            for s_i in range(CH):
                pltpu.make_async_copy(
                    table_hbm.at[idx_v.at[0]], bufs[b].at[s_i], gsems[b]
                ).wait()

        def start_wb(chunk, b):
            pltpu.async_copy(
                bufs[b], out_hbm.at[pl.ds(seq_base + chunk * CH, CH)], wsems[b])

        def wait_wb(b):
            pltpu.make_async_copy(
                bufs[b], out_hbm.at[pl.ds(0, CH)], wsems[b]).wait()

        def add_pos(b):
            buf = bufs[b]

            def add_body(l, c2):
                for s_i in range(CH):
                    for c4 in range(D // LANES):
                        col = c4 * LANES
                        plsc.addupdate(buf.at[s_i, l, pl.ds(col, LANES)],
                                       pos_v[l, pl.ds(col, LANES)])
                return c2

            lax.fori_loop(0, L, add_body, 0, unroll=2)

        start_gather(0, 0)
        start_gather(1, 1)
        wait_gather(0)
        add_pos(0)
        start_wb(0, 0)

        def outer(io, carry):
            for k in range(2):
                chunk = 2 * io + 1 + k
                b = (1 + k) % 2
                wait_gather(b)
                wait_wb(1 - b)
                start_gather(chunk + 1, 1 - b)
                add_pos(b)
                start_wb(chunk, b)
            return carry

        lax.fori_loop(0, (n_chunks - 2) // 2, outer, 0)

        wait_gather(1)
        wait_wb(0)
        add_pos(1)
        start_wb(n_chunks - 1, 1)
        wait_wb(1)

    return emb_kernel(x, embedding_table, pos_table)
